# baseline (speedup 1.0000x reference)
import jax
import jax.numpy as jnp
from jax.experimental import pallas as pl


def _softplus(x):
    return jnp.logaddexp(x, 0.0)


def _batchnorm(x, gamma, beta, eps=1e-5):
    mu = jnp.mean(x, axis=0, keepdims=True)
    var = jnp.var(x, axis=0, keepdims=True)
    return gamma * (x - mu) / jnp.sqrt(var + eps) + beta


def _fc_kernel(feats_ref, wfc_ref, bfc_ref, wout_ref, bout_ref, out_ref):
    f = _softplus(feats_ref[...])
    f = _softplus(f @ wfc_ref[...] + bfc_ref[...])
    f = _softplus(f)
    out_ref[...] = f @ wout_ref[...] + bout_ref[...]


def kernel(atom_features, bondlength, W_emb, b_emb, W_int, b_int, g_int, be_int, W_upd, b_upd, g_upd, be_upd, g_bn, be_bn, W_fc, b_fc, W_out, b_out, edge_index):
    DE = 16
    src = edge_index[0]
    dst = edge_index[1]
    centers = jnp.linspace(0.0, 8.0, DE)
    lengthscale = jnp.diff(centers).mean()
    gamma_rbf = 1.0 / lengthscale
    ef = jnp.exp(-gamma_rbf * (bondlength[:, None] - centers[None, :]) ** 2)
    h = atom_features @ W_emb + b_emb
    L = W_int.shape[0]
    for l in range(L):
        z = jnp.concatenate([h[src], h[dst], ef], axis=1)
        gate = jax.nn.sigmoid(_batchnorm(z @ W_int[l] + b_int[l], g_int[l], be_int[l]))
        upd = _softplus(_batchnorm(z @ W_upd[l] + b_upd[l], g_upd[l], be_upd[l]))
        m = gate * upd
        agg = jnp.zeros((h.shape[0], h.shape[1]), dtype=m.dtype).at[dst].add(m)
        h = _softplus(h + _batchnorm(agg, g_bn[l], be_bn[l]))
    feats = jnp.mean(h, axis=0, keepdims=True)
    out = pl.pallas_call(
        _fc_kernel,
        out_shape=jax.ShapeDtypeStruct((1, 1), jnp.float32),
    )(feats, W_fc, b_fc[None, :], W_out, b_out[None, :])
    return jnp.squeeze(out)


# speedup vs baseline: 1.7125x; 1.7125x over previous
import functools

import numpy as np
import jax
import jax.numpy as jnp
from jax import lax
from jax.experimental import pallas as pl
from jax.experimental.pallas import tpu as pltpu
from jax.experimental.pallas import tpu_sc as plsc

_N = 50000
_E = 800000
_D = 64
_DE = 16
_EPS = 1e-5

_BE = 4000
_BN = 2000

_NC = 2
_NS = 16
_NW = _NC * _NS
_GC = 200
_MC = 200
_SN = _N // _NS
_ZR = 625

_CENTERS = np.linspace(0.0, 8.0, _DE, dtype=np.float32)
_GAMMA = float(1.0 / np.diff(_CENTERS).mean())

@functools.lru_cache(maxsize=None)
def _sc_mesh():
    return plsc.VectorSubcoreMesh(core_axis_name="c", subcore_axis_name="s",
                                  num_cores=_NC, num_subcores=_NS)


def _softplus(x):
    return jnp.maximum(x, 0.0) + jnp.log(1.0 + jnp.exp(-jnp.abs(x)))


def _sigmoid(x):
    return 1.0 / (1.0 + jnp.exp(-x))


def _dot(a, b):
    return jax.lax.dot_general(a, b, (((1,), (0,)), ((), ())),
                               preferred_element_type=jnp.float32)


def _embed_body(a_ref, w_ref, b_ref, o_ref):
    o_ref[...] = _dot(a_ref[...], w_ref[...]) + b_ref[...]


def _edge_pre(ghs, ghd, bond, cen, ws_i, wd_i, we_i, ws_u, wd_u, we_u):
    d = bond[...] - cen[...]
    ef = jnp.exp(-_GAMMA * d * d)
    xi = _dot(ghs[...], ws_i[...]) + _dot(ghd[...], wd_i[...]) + _dot(ef, we_i[...])
    xu = _dot(ghs[...], ws_u[...]) + _dot(ghd[...], wd_u[...]) + _dot(ef, we_u[...])
    return xi, xu


def _edge_stats_body(ghs, ghd, bond, cen, ws_i, wd_i, we_i, ws_u, wd_u, we_u,
                     out_ref):
    xi, xu = _edge_pre(ghs, ghd, bond, cen, ws_i, wd_i, we_i, ws_u, wd_u, we_u)
    s = jnp.concatenate([jnp.sum(xi, 0, keepdims=True),
                         jnp.sum(xu, 0, keepdims=True)], axis=1)
    ss = jnp.concatenate([jnp.sum(xi * xi, 0, keepdims=True),
                          jnp.sum(xu * xu, 0, keepdims=True)], axis=1)
    val = jnp.concatenate([s, ss], axis=0)

    @pl.when(pl.program_id(0) == 0)
    def _():
        out_ref[...] = jnp.zeros_like(out_ref)

    out_ref[...] += val


def _edge_apply_body(ghs, ghd, bond, cen, ws_i, wd_i, we_i, ws_u, wd_u, we_u,
                     k_i, t_i, k_u, t_u, mlo_ref, mhi_ref):
    xi, xu = _edge_pre(ghs, ghd, bond, cen, ws_i, wd_i, we_i, ws_u, wd_u, we_u)
    gate = _sigmoid(xi * k_i[...] + t_i[...])
    upd = _softplus(xu * k_u[...] + t_u[...])
    m = gate * upd
    mlo_ref[...] = m[:, :32]
    mhi_ref[...] = m[:, 32:]


def _agg_stats_body(alo, ahi, out_ref):
    a = jnp.concatenate([alo[...], ahi[...]], axis=1)
    s = jnp.sum(a, 0, keepdims=True)
    ss = jnp.sum(a * a, 0, keepdims=True)
    val = jnp.concatenate([s, ss], axis=0)

    @pl.when(pl.program_id(0) == 0)
    def _():
        out_ref[...] = jnp.zeros_like(out_ref)

    out_ref[...] += val


def _update_body(h_ref, alo, ahi, k_ref, t_ref, hn_ref, hsum_ref):
    a = jnp.concatenate([alo[...], ahi[...]], axis=1)
    hn = _softplus(h_ref[...] + a * k_ref[...] + t_ref[...])
    hn_ref[...] = hn

    @pl.when(pl.program_id(0) == 0)
    def _():
        hsum_ref[...] = jnp.zeros_like(hsum_ref)

    hsum_ref[...] += jnp.sum(hn, 0, keepdims=True)


def _fc_body(hsum_ref, wfc_ref, bfc_ref, wout_ref, bout_ref, out_ref):
    f = _softplus(hsum_ref[...] * (1.0 / _N))
    f = _softplus(_dot(f, wfc_ref[...]) + bfc_ref[...])
    f = _softplus(f)
    out_ref[...] = _dot(f, wout_ref[...]) + bout_ref[...]


def _sc_gather_body(h_hbm, src_hbm, dst_hbm, os_hbm, od_hbm,
                    idx_s, idx_d, buf_s, buf_d, sem_s, sem_d):
    cid = lax.axis_index("c")
    sid = lax.axis_index("s")
    wid = sid * _NC + cid
    per_w = _E // _NW
    base = wid * per_w

    def step(i, carry):
        off = base + i * _GC
        pltpu.sync_copy(src_hbm.at[pl.ds(off, _GC)], idx_s)
        pltpu.sync_copy(dst_hbm.at[pl.ds(off, _GC)], idx_d)
        ca = pltpu.async_copy(h_hbm.at[idx_s], buf_s, sem_s)
        cb = pltpu.async_copy(h_hbm.at[idx_d], buf_d, sem_d)
        ca.wait()
        cb.wait()
        pltpu.sync_copy(buf_s, os_hbm.at[pl.ds(off, _GC), :])
        pltpu.sync_copy(buf_d, od_hbm.at[pl.ds(off, _GC), :])
        return carry

    lax.fori_loop(0, per_w // _GC, step, 0)


@functools.lru_cache(maxsize=None)
def _sc_gather_kernel():
    return pl.kernel(
        _sc_gather_body,
        out_type=[jax.ShapeDtypeStruct((_E, _D), jnp.float32),
                  jax.ShapeDtypeStruct((_E, _D), jnp.float32)],
        mesh=_sc_mesh(),
        scratch_types=[pltpu.VMEM((_GC,), jnp.int32),
                       pltpu.VMEM((_GC,), jnp.int32),
                       pltpu.VMEM((_GC, _D), jnp.float32),
                       pltpu.VMEM((_GC, _D), jnp.float32),
                       pltpu.SemaphoreType.DMA,
                       pltpu.SemaphoreType.DMA],
        compiler_params=pltpu.CompilerParams(use_tc_tiling_on_sc=False),
    )


def _sc_gather(h, src, dst):
    return _sc_gather_kernel()(h, src, dst)


def _sc_scatter_body(mlo_hbm, mhi_hbm, dst_hbm, olo_hbm, ohi_hbm,
                     idx_v, mbuf, zbuf, agg_sh):
    cid = lax.axis_index("c")
    sid = lax.axis_index("s")

    zv = jnp.zeros((16,), jnp.float32)

    def zstep(r, carry):
        zbuf[r, pl.ds(0, 16)] = zv
        zbuf[r, pl.ds(16, 16)] = zv
        return carry

    lax.fori_loop(0, _ZR, zstep, 0)

    def zcopy(j, carry):
        pltpu.sync_copy(zbuf, agg_sh.at[pl.ds(sid * _SN + j * _ZR, _ZR), :])
        return carry

    lax.fori_loop(0, _SN // _ZR, zcopy, 0)
    plsc.subcore_barrier()

    per_t = _E // _NS

    def run_half(m_hbm):
        def step(i, carry):
            off = sid * per_t + i * _MC
            pltpu.sync_copy(dst_hbm.at[pl.ds(off, _MC)], idx_v)
            pltpu.sync_copy(m_hbm.at[pl.ds(off, _MC), :], mbuf)
            pltpu.sync_copy(mbuf, agg_sh.at[idx_v], add=True)
            return carry

        lax.fori_loop(0, per_t // _MC, step, 0)

    @pl.when(cid == 0)
    def _():
        run_half(mlo_hbm)

    @pl.when(cid == 1)
    def _():
        run_half(mhi_hbm)

    plsc.subcore_barrier()

    @pl.when(cid == 0)
    def _():
        pltpu.sync_copy(agg_sh.at[pl.ds(sid * _SN, _SN), :],
                        olo_hbm.at[pl.ds(sid * _SN, _SN), :])

    @pl.when(cid == 1)
    def _():
        pltpu.sync_copy(agg_sh.at[pl.ds(sid * _SN, _SN), :],
                        ohi_hbm.at[pl.ds(sid * _SN, _SN), :])


@functools.lru_cache(maxsize=None)
def _sc_scatter_kernel():
    return pl.kernel(
        _sc_scatter_body,
        out_type=[jax.ShapeDtypeStruct((_N, 32), jnp.float32),
                  jax.ShapeDtypeStruct((_N, 32), jnp.float32)],
        mesh=_sc_mesh(),
        scratch_types=[pltpu.VMEM((_MC,), jnp.int32),
                       pltpu.VMEM((_MC, 32), jnp.float32),
                       pltpu.VMEM((_ZR, 32), jnp.float32),
                       pltpu.VMEM_SHARED((_N, 32), jnp.float32)],
        compiler_params=pltpu.CompilerParams(use_tc_tiling_on_sc=False),
    )


def _sc_scatter(mlo, mhi, dst):
    return _sc_scatter_kernel()(mlo, mhi, dst)


def _tc_embed(atom, w, b):
    steps = _N // _BN
    return pl.pallas_call(
        _embed_body,
        grid=(steps,),
        in_specs=[pl.BlockSpec((_BN, 92), lambda i: (i, 0)),
                  pl.BlockSpec((92, _D), lambda i: (0, 0)),
                  pl.BlockSpec((1, _D), lambda i: (0, 0))],
        out_specs=pl.BlockSpec((_BN, _D), lambda i: (i, 0)),
        out_shape=jax.ShapeDtypeStruct((_N, _D), jnp.float32),
    )(atom, w, b)


def _edge_in_specs():
    return [pl.BlockSpec((_BE, _D), lambda i: (i, 0)),
            pl.BlockSpec((_BE, _D), lambda i: (i, 0)),
            pl.BlockSpec((_BE, 1), lambda i: (i, 0)),
            pl.BlockSpec((1, _DE), lambda i: (0, 0)),
            pl.BlockSpec((_D, _D), lambda i: (0, 0)),
            pl.BlockSpec((_D, _D), lambda i: (0, 0)),
            pl.BlockSpec((_DE, _D), lambda i: (0, 0)),
            pl.BlockSpec((_D, _D), lambda i: (0, 0)),
            pl.BlockSpec((_D, _D), lambda i: (0, 0)),
            pl.BlockSpec((_DE, _D), lambda i: (0, 0))]


def _tc_edge_stats(ghs, ghd, bond2d, cen, ws_i, wd_i, we_i, ws_u, wd_u, we_u):
    steps = _E // _BE
    return pl.pallas_call(
        _edge_stats_body,
        grid=(steps,),
        in_specs=_edge_in_specs(),
        out_specs=pl.BlockSpec((2, 2 * _D), lambda i: (0, 0)),
        out_shape=jax.ShapeDtypeStruct((2, 2 * _D), jnp.float32),
    )(ghs, ghd, bond2d, cen, ws_i, wd_i, we_i, ws_u, wd_u, we_u)


def _tc_edge_apply(ghs, ghd, bond2d, cen, ws_i, wd_i, we_i, ws_u, wd_u, we_u,
                   k_i, t_i, k_u, t_u):
    steps = _E // _BE
    vec = pl.BlockSpec((1, _D), lambda i: (0, 0))
    return pl.pallas_call(
        _edge_apply_body,
        grid=(steps,),
        in_specs=_edge_in_specs() + [vec, vec, vec, vec],
        out_specs=[pl.BlockSpec((_BE, 32), lambda i: (i, 0)),
                   pl.BlockSpec((_BE, 32), lambda i: (i, 0))],
        out_shape=[jax.ShapeDtypeStruct((_E, 32), jnp.float32),
                   jax.ShapeDtypeStruct((_E, 32), jnp.float32)],
    )(ghs, ghd, bond2d, cen, ws_i, wd_i, we_i, ws_u, wd_u, we_u,
      k_i, t_i, k_u, t_u)


def _tc_agg_stats(alo, ahi):
    steps = _N // _BN
    return pl.pallas_call(
        _agg_stats_body,
        grid=(steps,),
        in_specs=[pl.BlockSpec((_BN, 32), lambda i: (i, 0)),
                  pl.BlockSpec((_BN, 32), lambda i: (i, 0))],
        out_specs=pl.BlockSpec((2, _D), lambda i: (0, 0)),
        out_shape=jax.ShapeDtypeStruct((2, _D), jnp.float32),
    )(alo, ahi)


def _tc_update(h, alo, ahi, k, t):
    steps = _N // _BN
    return pl.pallas_call(
        _update_body,
        grid=(steps,),
        in_specs=[pl.BlockSpec((_BN, _D), lambda i: (i, 0)),
                  pl.BlockSpec((_BN, 32), lambda i: (i, 0)),
                  pl.BlockSpec((_BN, 32), lambda i: (i, 0)),
                  pl.BlockSpec((1, _D), lambda i: (0, 0)),
                  pl.BlockSpec((1, _D), lambda i: (0, 0))],
        out_specs=[pl.BlockSpec((_BN, _D), lambda i: (i, 0)),
                   pl.BlockSpec((1, _D), lambda i: (0, 0))],
        out_shape=[jax.ShapeDtypeStruct((_N, _D), jnp.float32),
                   jax.ShapeDtypeStruct((1, _D), jnp.float32)],
    )(h, alo, ahi, k, t)


def _tc_fc(hsum, wfc, bfc, wout, bout):
    return pl.pallas_call(
        _fc_body,
        out_shape=jax.ShapeDtypeStruct((1, 1), jnp.float32),
    )(hsum, wfc, bfc[None, :], wout, bout[None, :])


def _affine(g, be, sums, n):
    s, ss = sums[0], sums[1]
    mu = s / n
    var = ss / n - mu * mu
    rstd = 1.0 / jnp.sqrt(var + _EPS)
    k = g * rstd
    t = be - g * mu * rstd
    return k[None, :], t[None, :]


def kernel(atom_features, bondlength, W_emb, b_emb, W_int, b_int, g_int,
           be_int, W_upd, b_upd, g_upd, be_upd, g_bn, be_bn, W_fc, b_fc,
           W_out, b_out, edge_index):
    ei = edge_index.astype(jnp.int32)
    src = ei[0]
    dst = ei[1]
    bond2d = bondlength.reshape(_E, 1)
    cen = jnp.asarray(_CENTERS)[None, :]

    h = _tc_embed(atom_features, W_emb, b_emb[None, :])

    L = W_int.shape[0]
    hsum = None
    for l in range(L):
        ws_i, wd_i, we_i = W_int[l, :_D], W_int[l, _D:2 * _D], W_int[l, 2 * _D:]
        ws_u, wd_u, we_u = W_upd[l, :_D], W_upd[l, _D:2 * _D], W_upd[l, 2 * _D:]

        ghs, ghd = _sc_gather(h, src, dst)
        stats = _tc_edge_stats(ghs, ghd, bond2d, cen,
                               ws_i, wd_i, we_i, ws_u, wd_u, we_u)
        k_cat, t_cat = _affine(jnp.concatenate([g_int[l], g_upd[l]]),
                               jnp.concatenate([be_int[l], be_upd[l]]),
                               stats, float(_E))
        k_i, k_u = k_cat[:, :_D], k_cat[:, _D:]
        t_i, t_u = t_cat[:, :_D], t_cat[:, _D:]
        mlo, mhi = _tc_edge_apply(ghs, ghd, bond2d, cen,
                                  ws_i, wd_i, we_i, ws_u, wd_u, we_u,
                                  k_i, t_i, k_u, t_u)
        alo, ahi = _sc_scatter(mlo, mhi, dst)
        astats = _tc_agg_stats(alo, ahi)
        k_bn, t_bn = _affine(g_bn[l], be_bn[l], astats, float(_N))
        h, hsum = _tc_update(h, alo, ahi, k_bn, t_bn)

    out = _tc_fc(hsum, W_fc, b_fc, W_out, b_out)
    return jnp.squeeze(out)


# speedup vs baseline: 1.7676x; 1.0322x over previous
import functools

import numpy as np
import jax
import jax.numpy as jnp
from jax import lax
from jax.experimental import pallas as pl
from jax.experimental.pallas import tpu as pltpu
from jax.experimental.pallas import tpu_sc as plsc

_N = 50000
_E = 800000
_D = 64
_DE = 16
_EPS = 1e-5

_BE = 4000
_BN = 2000

_NC = 2
_NS = 16
_NW = _NC * _NS
_GC = 1000
_MC = 400
_SN = _N // _NS
_ZR = 125

_CENTERS = np.linspace(0.0, 8.0, _DE, dtype=np.float32)
_GAMMA = float(1.0 / np.diff(_CENTERS).mean())

@functools.lru_cache(maxsize=None)
def _sc_mesh():
    return plsc.VectorSubcoreMesh(core_axis_name="c", subcore_axis_name="s",
                                  num_cores=_NC, num_subcores=_NS)


def _softplus(x):
    return jnp.maximum(x, 0.0) + jnp.log(1.0 + jnp.exp(-jnp.abs(x)))


def _sigmoid(x):
    return 1.0 / (1.0 + jnp.exp(-x))


def _dot(a, b):
    return jax.lax.dot_general(a, b, (((1,), (0,)), ((), ())),
                               preferred_element_type=jnp.float32)


def _embed_body(a_ref, w_ref, b_ref, o_ref, ob_ref):
    h = _dot(a_ref[...], w_ref[...]) + b_ref[...]
    o_ref[...] = h
    ob_ref[...] = h.astype(jnp.bfloat16)


def _edge_pre(ghs, ghd, bond, cen, ws_i, wd_i, we_i, ws_u, wd_u, we_u):
    d = bond[...] - cen[...]
    ef = jnp.exp(-_GAMMA * d * d)
    xi = _dot(ghs[...], ws_i[...]) + _dot(ghd[...], wd_i[...]) + _dot(ef, we_i[...])
    xu = _dot(ghs[...], ws_u[...]) + _dot(ghd[...], wd_u[...]) + _dot(ef, we_u[...])
    return xi, xu


def _edge_stats_body(ghs, ghd, bond, cen, ws_i, wd_i, we_i, ws_u, wd_u, we_u,
                     out_ref):
    xi, xu = _edge_pre(ghs, ghd, bond, cen, ws_i, wd_i, we_i, ws_u, wd_u, we_u)
    s = jnp.concatenate([jnp.sum(xi, 0, keepdims=True),
                         jnp.sum(xu, 0, keepdims=True)], axis=1)
    ss = jnp.concatenate([jnp.sum(xi * xi, 0, keepdims=True),
                          jnp.sum(xu * xu, 0, keepdims=True)], axis=1)
    val = jnp.concatenate([s, ss], axis=0)

    @pl.when(pl.program_id(0) == 0)
    def _():
        out_ref[...] = jnp.zeros_like(out_ref)

    out_ref[...] += val


def _edge_apply_body(ghs, ghd, bond, cen, ws_i, wd_i, we_i, ws_u, wd_u, we_u,
                     k_i, t_i, k_u, t_u, mlo_ref, mhi_ref):
    xi, xu = _edge_pre(ghs, ghd, bond, cen, ws_i, wd_i, we_i, ws_u, wd_u, we_u)
    gate = _sigmoid(xi * k_i[...] + t_i[...])
    upd = _softplus(xu * k_u[...] + t_u[...])
    m = gate * upd
    mlo_ref[...] = m[:, :32]
    mhi_ref[...] = m[:, 32:]


def _agg_stats_body(alo, ahi, out_ref):
    a = jnp.concatenate([alo[...], ahi[...]], axis=1)
    s = jnp.sum(a, 0, keepdims=True)
    ss = jnp.sum(a * a, 0, keepdims=True)
    val = jnp.concatenate([s, ss], axis=0)

    @pl.when(pl.program_id(0) == 0)
    def _():
        out_ref[...] = jnp.zeros_like(out_ref)

    out_ref[...] += val


def _update_body(h_ref, alo, ahi, k_ref, t_ref, hn_ref, hnb_ref, hsum_ref):
    a = jnp.concatenate([alo[...], ahi[...]], axis=1)
    hn = _softplus(h_ref[...] + a * k_ref[...] + t_ref[...])
    hn_ref[...] = hn
    hnb_ref[...] = hn.astype(jnp.bfloat16)

    @pl.when(pl.program_id(0) == 0)
    def _():
        hsum_ref[...] = jnp.zeros_like(hsum_ref)

    hsum_ref[...] += jnp.sum(hn, 0, keepdims=True)


def _fc_body(hsum_ref, wfc_ref, bfc_ref, wout_ref, bout_ref, out_ref):
    f = _softplus(hsum_ref[...] * (1.0 / _N))
    f = _softplus(_dot(f, wfc_ref[...]) + bfc_ref[...])
    f = _softplus(f)
    out_ref[...] = _dot(f, wout_ref[...]) + bout_ref[...]


def _sc_gather_body(h_hbm, src_hbm, dst_hbm, os_hbm, od_hbm,
                    idx_s, idx_d, buf_s, buf_d, sem_s, sem_d):
    cid = lax.axis_index("c")
    sid = lax.axis_index("s")
    wid = sid * _NC + cid
    per_w = _E // _NW
    base = wid * per_w

    def step(i, carry):
        off = base + i * _GC
        pltpu.sync_copy(src_hbm.at[pl.ds(off, _GC)], idx_s)
        pltpu.sync_copy(dst_hbm.at[pl.ds(off, _GC)], idx_d)
        ca = pltpu.async_copy(h_hbm.at[idx_s], buf_s, sem_s)
        cb = pltpu.async_copy(h_hbm.at[idx_d], buf_d, sem_d)
        ca.wait()
        cb.wait()
        pltpu.sync_copy(buf_s, os_hbm.at[pl.ds(off, _GC), :])
        pltpu.sync_copy(buf_d, od_hbm.at[pl.ds(off, _GC), :])
        return carry

    lax.fori_loop(0, per_w // _GC, step, 0)


@functools.lru_cache(maxsize=None)
def _sc_gather_kernel():
    return pl.kernel(
        _sc_gather_body,
        out_type=[jax.ShapeDtypeStruct((_E, _D), jnp.bfloat16),
                  jax.ShapeDtypeStruct((_E, _D), jnp.bfloat16)],
        mesh=_sc_mesh(),
        scratch_types=[pltpu.VMEM((_GC,), jnp.int32),
                       pltpu.VMEM((_GC,), jnp.int32),
                       pltpu.VMEM((_GC, _D), jnp.bfloat16),
                       pltpu.VMEM((_GC, _D), jnp.bfloat16),
                       pltpu.SemaphoreType.DMA,
                       pltpu.SemaphoreType.DMA],
        compiler_params=pltpu.CompilerParams(use_tc_tiling_on_sc=False),
    )


def _sc_gather(h, src, dst):
    return _sc_gather_kernel()(h, src, dst)


def _sc_scatter_body(mlo_hbm, mhi_hbm, dst_hbm, olo_hbm, ohi_hbm,
                     idx_v, mbuf, zbuf, agg_sh):
    cid = lax.axis_index("c")
    sid = lax.axis_index("s")

    zv = jnp.zeros((16,), jnp.float32)

    def zstep(r, carry):
        zbuf[r, pl.ds(0, 16)] = zv
        zbuf[r, pl.ds(16, 16)] = zv
        return carry

    lax.fori_loop(0, _ZR, zstep, 0)

    def zcopy(j, carry):
        pltpu.sync_copy(zbuf, agg_sh.at[pl.ds(sid * _SN + j * _ZR, _ZR), :])
        return carry

    lax.fori_loop(0, _SN // _ZR, zcopy, 0)
    plsc.subcore_barrier()

    per_t = _E // _NS

    def run_half(m_hbm):
        def step(i, carry):
            off = sid * per_t + i * _MC
            pltpu.sync_copy(dst_hbm.at[pl.ds(off, _MC)], idx_v)
            pltpu.sync_copy(m_hbm.at[pl.ds(off, _MC), :], mbuf)
            pltpu.sync_copy(mbuf, agg_sh.at[idx_v], add=True)
            return carry

        lax.fori_loop(0, per_t // _MC, step, 0)

    @pl.when(cid == 0)
    def _():
        run_half(mlo_hbm)

    @pl.when(cid == 1)
    def _():
        run_half(mhi_hbm)

    plsc.subcore_barrier()

    @pl.when(cid == 0)
    def _():
        pltpu.sync_copy(agg_sh.at[pl.ds(sid * _SN, _SN), :],
                        olo_hbm.at[pl.ds(sid * _SN, _SN), :])

    @pl.when(cid == 1)
    def _():
        pltpu.sync_copy(agg_sh.at[pl.ds(sid * _SN, _SN), :],
                        ohi_hbm.at[pl.ds(sid * _SN, _SN), :])


@functools.lru_cache(maxsize=None)
def _sc_scatter_kernel():
    return pl.kernel(
        _sc_scatter_body,
        out_type=[jax.ShapeDtypeStruct((_N, 32), jnp.float32),
                  jax.ShapeDtypeStruct((_N, 32), jnp.float32)],
        mesh=_sc_mesh(),
        scratch_types=[pltpu.VMEM((_MC,), jnp.int32),
                       pltpu.VMEM((_MC, 32), jnp.float32),
                       pltpu.VMEM((_ZR, 32), jnp.float32),
                       pltpu.VMEM_SHARED((_N, 32), jnp.float32)],
        compiler_params=pltpu.CompilerParams(use_tc_tiling_on_sc=False),
    )


def _sc_scatter(mlo, mhi, dst):
    return _sc_scatter_kernel()(mlo, mhi, dst)


def _tc_embed(atom, w, b):
    steps = _N // _BN
    return pl.pallas_call(
        _embed_body,
        grid=(steps,),
        in_specs=[pl.BlockSpec((_BN, 92), lambda i: (i, 0)),
                  pl.BlockSpec((92, _D), lambda i: (0, 0)),
                  pl.BlockSpec((1, _D), lambda i: (0, 0))],
        out_specs=[pl.BlockSpec((_BN, _D), lambda i: (i, 0)),
                   pl.BlockSpec((_BN, _D), lambda i: (i, 0))],
        out_shape=[jax.ShapeDtypeStruct((_N, _D), jnp.float32),
                   jax.ShapeDtypeStruct((_N, _D), jnp.bfloat16)],
    )(atom, w, b)


def _edge_in_specs():
    return [pl.BlockSpec((_BE, _D), lambda i: (i, 0)),
            pl.BlockSpec((_BE, _D), lambda i: (i, 0)),
            pl.BlockSpec((_BE, 1), lambda i: (i, 0)),
            pl.BlockSpec((1, _DE), lambda i: (0, 0)),
            pl.BlockSpec((_D, _D), lambda i: (0, 0)),
            pl.BlockSpec((_D, _D), lambda i: (0, 0)),
            pl.BlockSpec((_DE, _D), lambda i: (0, 0)),
            pl.BlockSpec((_D, _D), lambda i: (0, 0)),
            pl.BlockSpec((_D, _D), lambda i: (0, 0)),
            pl.BlockSpec((_DE, _D), lambda i: (0, 0))]


def _tc_edge_stats(ghs, ghd, bond2d, cen, ws_i, wd_i, we_i, ws_u, wd_u, we_u):
    steps = _E // _BE
    return pl.pallas_call(
        _edge_stats_body,
        grid=(steps,),
        in_specs=_edge_in_specs(),
        out_specs=pl.BlockSpec((2, 2 * _D), lambda i: (0, 0)),
        out_shape=jax.ShapeDtypeStruct((2, 2 * _D), jnp.float32),
    )(ghs, ghd, bond2d, cen, ws_i, wd_i, we_i, ws_u, wd_u, we_u)


def _tc_edge_apply(ghs, ghd, bond2d, cen, ws_i, wd_i, we_i, ws_u, wd_u, we_u,
                   k_i, t_i, k_u, t_u):
    steps = _E // _BE
    vec = pl.BlockSpec((1, _D), lambda i: (0, 0))
    return pl.pallas_call(
        _edge_apply_body,
        grid=(steps,),
        in_specs=_edge_in_specs() + [vec, vec, vec, vec],
        out_specs=[pl.BlockSpec((_BE, 32), lambda i: (i, 0)),
                   pl.BlockSpec((_BE, 32), lambda i: (i, 0))],
        out_shape=[jax.ShapeDtypeStruct((_E, 32), jnp.float32),
                   jax.ShapeDtypeStruct((_E, 32), jnp.float32)],
    )(ghs, ghd, bond2d, cen, ws_i, wd_i, we_i, ws_u, wd_u, we_u,
      k_i, t_i, k_u, t_u)


def _tc_agg_stats(alo, ahi):
    steps = _N // _BN
    return pl.pallas_call(
        _agg_stats_body,
        grid=(steps,),
        in_specs=[pl.BlockSpec((_BN, 32), lambda i: (i, 0)),
                  pl.BlockSpec((_BN, 32), lambda i: (i, 0))],
        out_specs=pl.BlockSpec((2, _D), lambda i: (0, 0)),
        out_shape=jax.ShapeDtypeStruct((2, _D), jnp.float32),
    )(alo, ahi)


def _tc_update(h, alo, ahi, k, t):
    steps = _N // _BN
    return pl.pallas_call(
        _update_body,
        grid=(steps,),
        in_specs=[pl.BlockSpec((_BN, _D), lambda i: (i, 0)),
                  pl.BlockSpec((_BN, 32), lambda i: (i, 0)),
                  pl.BlockSpec((_BN, 32), lambda i: (i, 0)),
                  pl.BlockSpec((1, _D), lambda i: (0, 0)),
                  pl.BlockSpec((1, _D), lambda i: (0, 0))],
        out_specs=[pl.BlockSpec((_BN, _D), lambda i: (i, 0)),
                   pl.BlockSpec((_BN, _D), lambda i: (i, 0)),
                   pl.BlockSpec((1, _D), lambda i: (0, 0))],
        out_shape=[jax.ShapeDtypeStruct((_N, _D), jnp.float32),
                   jax.ShapeDtypeStruct((_N, _D), jnp.bfloat16),
                   jax.ShapeDtypeStruct((1, _D), jnp.float32)],
    )(h, alo, ahi, k, t)


def _tc_fc(hsum, wfc, bfc, wout, bout):
    return pl.pallas_call(
        _fc_body,
        out_shape=jax.ShapeDtypeStruct((1, 1), jnp.float32),
    )(hsum, wfc, bfc[None, :], wout, bout[None, :])


def _affine(g, be, sums, n):
    s, ss = sums[0], sums[1]
    mu = s / n
    var = ss / n - mu * mu
    rstd = 1.0 / jnp.sqrt(var + _EPS)
    k = g * rstd
    t = be - g * mu * rstd
    return k[None, :], t[None, :]


def kernel(atom_features, bondlength, W_emb, b_emb, W_int, b_int, g_int,
           be_int, W_upd, b_upd, g_upd, be_upd, g_bn, be_bn, W_fc, b_fc,
           W_out, b_out, edge_index):
    ei = edge_index.astype(jnp.int32)
    src = ei[0]
    dst = ei[1]
    bond2d = bondlength.reshape(_E, 1)
    cen = jnp.asarray(_CENTERS)[None, :]

    h, hb = _tc_embed(atom_features, W_emb, b_emb[None, :])

    bf = jnp.bfloat16
    L = W_int.shape[0]
    hsum = None
    for l in range(L):
        ws_i, wd_i = W_int[l, :_D].astype(bf), W_int[l, _D:2 * _D].astype(bf)
        ws_u, wd_u = W_upd[l, :_D].astype(bf), W_upd[l, _D:2 * _D].astype(bf)
        we_i, we_u = W_int[l, 2 * _D:], W_upd[l, 2 * _D:]

        ghs, ghd = _sc_gather(hb, src, dst)
        stats = _tc_edge_stats(ghs, ghd, bond2d, cen,
                               ws_i, wd_i, we_i, ws_u, wd_u, we_u)
        k_cat, t_cat = _affine(jnp.concatenate([g_int[l], g_upd[l]]),
                               jnp.concatenate([be_int[l], be_upd[l]]),
                               stats, float(_E))
        k_i, k_u = k_cat[:, :_D], k_cat[:, _D:]
        t_i, t_u = t_cat[:, :_D], t_cat[:, _D:]
        mlo, mhi = _tc_edge_apply(ghs, ghd, bond2d, cen,
                                  ws_i, wd_i, we_i, ws_u, wd_u, we_u,
                                  k_i, t_i, k_u, t_u)
        alo, ahi = _sc_scatter(mlo, mhi, dst)
        astats = _tc_agg_stats(alo, ahi)
        k_bn, t_bn = _affine(g_bn[l], be_bn[l], astats, float(_N))
        h, hb, hsum = _tc_update(h, alo, ahi, k_bn, t_bn)

    out = _tc_fc(hsum, W_fc, b_fc, W_out, b_out)
    return jnp.squeeze(out)


# speedup vs baseline: 1.9643x; 1.1113x over previous
import functools

import numpy as np
import jax
import jax.numpy as jnp
from jax import lax
from jax.experimental import pallas as pl
from jax.experimental.pallas import tpu as pltpu
from jax.experimental.pallas import tpu_sc as plsc

_N = 50000
_E = 800000
_D = 64
_DE = 16
_EPS = 1e-5

_BE = 4000
_BN = 2000

_NC = 2
_NS = 16
_NW = _NC * _NS
_GC = 1000
_MC = 400
_SN = _N // _NS
_ZR = 125

_CENTERS = np.linspace(0.0, 8.0, _DE, dtype=np.float32)
_GAMMA = float(1.0 / np.diff(_CENTERS).mean())

@functools.lru_cache(maxsize=None)
def _sc_mesh():
    return plsc.VectorSubcoreMesh(core_axis_name="c", subcore_axis_name="s",
                                  num_cores=_NC, num_subcores=_NS)


def _softplus(x):
    return jnp.maximum(x, 0.0) + jnp.log(1.0 + jnp.exp(-jnp.abs(x)))


def _sigmoid(x):
    return 1.0 / (1.0 + jnp.exp(-x))


def _dot(a, b):
    return jax.lax.dot_general(a, b, (((1,), (0,)), ((), ())),
                               preferred_element_type=jnp.float32)


def _embed_body(a_ref, w_ref, b_ref, o_ref, hsum_ref):
    h = _dot(a_ref[...], w_ref[...]) + b_ref[...]
    o_ref[...] = h

    @pl.when(pl.program_id(0) == 0)
    def _():
        hsum_ref[...] = jnp.zeros_like(hsum_ref)

    hsum_ref[...] += jnp.sum(h, 0, keepdims=True)


def _ef_body(bond_ref, cen_ref, ef_ref):
    d = bond_ref[...] - cen_ref[...]
    ef_ref[...] = jnp.exp(-_GAMMA * d * d).astype(jnp.bfloat16)


def _center_body(h_ref, mu_ref, hb_ref):
    hb_ref[...] = (h_ref[...] - mu_ref[...]).astype(jnp.bfloat16)


def _edge_stats_body(gh, efb, wcat, wef, out_ref):
    x = _dot(gh[...], wcat[...]) + _dot(efb[...], wef[...])
    s = jnp.sum(x, 0, keepdims=True)
    ss = jnp.sum(x * x, 0, keepdims=True)
    val = jnp.concatenate([s, ss], axis=0)

    @pl.when(pl.program_id(0) == 0)
    def _():
        out_ref[...] = jnp.zeros_like(out_ref)

    out_ref[...] += val


def _edge_apply_body(gh, efb, wcat, wef, k_ref, t_ref, mlo_ref, mhi_ref):
    x = _dot(gh[...], wcat[...]) + _dot(efb[...], wef[...])
    y = x * k_ref[...] + t_ref[...]
    gate = _sigmoid(y[:, :_D])
    upd = _softplus(y[:, _D:])
    m = gate * upd
    mlo_ref[...] = m[:, :32]
    mhi_ref[...] = m[:, 32:]


def _agg_stats_body(alo, ahi, out_ref):
    a = jnp.concatenate([alo[...], ahi[...]], axis=1)
    s = jnp.sum(a, 0, keepdims=True)
    ss = jnp.sum(a * a, 0, keepdims=True)
    val = jnp.concatenate([s, ss], axis=0)

    @pl.when(pl.program_id(0) == 0)
    def _():
        out_ref[...] = jnp.zeros_like(out_ref)

    out_ref[...] += val


def _update_body(h_ref, alo, ahi, k_ref, t_ref, hn_ref, hsum_ref):
    a = jnp.concatenate([alo[...], ahi[...]], axis=1)
    hn = _softplus(h_ref[...] + a * k_ref[...] + t_ref[...])
    hn_ref[...] = hn

    @pl.when(pl.program_id(0) == 0)
    def _():
        hsum_ref[...] = jnp.zeros_like(hsum_ref)

    hsum_ref[...] += jnp.sum(hn, 0, keepdims=True)


def _fc_body(hsum_ref, wfc_ref, bfc_ref, wout_ref, bout_ref, out_ref):
    f = _softplus(hsum_ref[...] * (1.0 / _N))
    f = _softplus(_dot(f, wfc_ref[...]) + bfc_ref[...])
    f = _softplus(f)
    out_ref[...] = _dot(f, wout_ref[...]) + bout_ref[...]


def _sc_gather_body(h_hbm, src_hbm, dst_hbm, gh_hbm,
                    idx_s, idx_d, buf_s, buf_d, sem_s, sem_d):
    cid = lax.axis_index("c")
    sid = lax.axis_index("s")
    wid = sid * _NC + cid
    per_w = _E // _NW
    base = wid * per_w

    def step(i, carry):
        off = base + i * _GC
        pltpu.sync_copy(src_hbm.at[pl.ds(off, _GC)], idx_s)
        pltpu.sync_copy(dst_hbm.at[pl.ds(off, _GC)], idx_d)
        ca = pltpu.async_copy(h_hbm.at[idx_s], buf_s, sem_s)
        cb = pltpu.async_copy(h_hbm.at[idx_d], buf_d, sem_d)
        ca.wait()
        cb.wait()
        pltpu.sync_copy(buf_s, gh_hbm.at[pl.ds(off, _GC), pl.ds(0, _D)])
        pltpu.sync_copy(buf_d, gh_hbm.at[pl.ds(off, _GC), pl.ds(_D, _D)])
        return carry

    lax.fori_loop(0, per_w // _GC, step, 0)


@functools.lru_cache(maxsize=None)
def _sc_gather_kernel():
    return pl.kernel(
        _sc_gather_body,
        out_type=[jax.ShapeDtypeStruct((_E, 2 * _D), jnp.bfloat16)],
        mesh=_sc_mesh(),
        scratch_types=[pltpu.VMEM((_GC,), jnp.int32),
                       pltpu.VMEM((_GC,), jnp.int32),
                       pltpu.VMEM((_GC, _D), jnp.bfloat16),
                       pltpu.VMEM((_GC, _D), jnp.bfloat16),
                       pltpu.SemaphoreType.DMA,
                       pltpu.SemaphoreType.DMA],
        compiler_params=pltpu.CompilerParams(use_tc_tiling_on_sc=False),
    )


def _sc_gather(h, src, dst):
    return _sc_gather_kernel()(h, src, dst)[0]


def _sc_scatter_body(mlo_hbm, mhi_hbm, dst_hbm, olo_hbm, ohi_hbm,
                     idx_v, mbuf, zbuf, agg_sh):
    cid = lax.axis_index("c")
    sid = lax.axis_index("s")

    zv = jnp.zeros((16,), jnp.float32)

    def zstep(r, carry):
        zbuf[r, pl.ds(0, 16)] = zv
        zbuf[r, pl.ds(16, 16)] = zv
        return carry

    lax.fori_loop(0, _ZR, zstep, 0)

    def zcopy(j, carry):
        pltpu.sync_copy(zbuf, agg_sh.at[pl.ds(sid * _SN + j * _ZR, _ZR), :])
        return carry

    lax.fori_loop(0, _SN // _ZR, zcopy, 0)
    plsc.subcore_barrier()

    per_t = _E // _NS

    def run_half(m_hbm):
        def step(i, carry):
            off = sid * per_t + i * _MC
            pltpu.sync_copy(dst_hbm.at[pl.ds(off, _MC)], idx_v)
            pltpu.sync_copy(m_hbm.at[pl.ds(off, _MC), :], mbuf)
            pltpu.sync_copy(mbuf, agg_sh.at[idx_v], add=True)
            return carry

        lax.fori_loop(0, per_t // _MC, step, 0)

    @pl.when(cid == 0)
    def _():
        run_half(mlo_hbm)

    @pl.when(cid == 1)
    def _():
        run_half(mhi_hbm)

    plsc.subcore_barrier()

    @pl.when(cid == 0)
    def _():
        pltpu.sync_copy(agg_sh.at[pl.ds(sid * _SN, _SN), :],
                        olo_hbm.at[pl.ds(sid * _SN, _SN), :])

    @pl.when(cid == 1)
    def _():
        pltpu.sync_copy(agg_sh.at[pl.ds(sid * _SN, _SN), :],
                        ohi_hbm.at[pl.ds(sid * _SN, _SN), :])


@functools.lru_cache(maxsize=None)
def _sc_scatter_kernel():
    return pl.kernel(
        _sc_scatter_body,
        out_type=[jax.ShapeDtypeStruct((_N, 32), jnp.float32),
                  jax.ShapeDtypeStruct((_N, 32), jnp.float32)],
        mesh=_sc_mesh(),
        scratch_types=[pltpu.VMEM((_MC,), jnp.int32),
                       pltpu.VMEM((_MC, 32), jnp.float32),
                       pltpu.VMEM((_ZR, 32), jnp.float32),
                       pltpu.VMEM_SHARED((_N, 32), jnp.float32)],
        compiler_params=pltpu.CompilerParams(use_tc_tiling_on_sc=False),
    )


def _sc_scatter(mlo, mhi, dst):
    return _sc_scatter_kernel()(mlo, mhi, dst)


def _tc_embed(atom, w, b):
    steps = _N // _BN
    return pl.pallas_call(
        _embed_body,
        grid=(steps,),
        in_specs=[pl.BlockSpec((_BN, 92), lambda i: (i, 0)),
                  pl.BlockSpec((92, _D), lambda i: (0, 0)),
                  pl.BlockSpec((1, _D), lambda i: (0, 0))],
        out_specs=[pl.BlockSpec((_BN, _D), lambda i: (i, 0)),
                   pl.BlockSpec((1, _D), lambda i: (0, 0))],
        out_shape=[jax.ShapeDtypeStruct((_N, _D), jnp.float32),
                   jax.ShapeDtypeStruct((1, _D), jnp.float32)],
    )(atom, w, b)


def _tc_ef(bond2d, cen):
    steps = _E // _BE
    return pl.pallas_call(
        _ef_body,
        grid=(steps,),
        in_specs=[pl.BlockSpec((_BE, 1), lambda i: (i, 0)),
                  pl.BlockSpec((1, _DE), lambda i: (0, 0))],
        out_specs=pl.BlockSpec((_BE, _DE), lambda i: (i, 0)),
        out_shape=jax.ShapeDtypeStruct((_E, _DE), jnp.bfloat16),
    )(bond2d, cen)


def _tc_center(h, mu):
    steps = _N // _BN
    return pl.pallas_call(
        _center_body,
        grid=(steps,),
        in_specs=[pl.BlockSpec((_BN, _D), lambda i: (i, 0)),
                  pl.BlockSpec((1, _D), lambda i: (0, 0))],
        out_specs=pl.BlockSpec((_BN, _D), lambda i: (i, 0)),
        out_shape=jax.ShapeDtypeStruct((_N, _D), jnp.bfloat16),
    )(h, mu)


def _edge_in_specs():
    return [pl.BlockSpec((_BE, 2 * _D), lambda i: (i, 0)),
            pl.BlockSpec((_BE, _DE), lambda i: (i, 0)),
            pl.BlockSpec((2 * _D, 2 * _D), lambda i: (0, 0)),
            pl.BlockSpec((_DE, 2 * _D), lambda i: (0, 0))]


def _tc_edge_stats(gh, efb, wcat, wef):
    steps = _E // _BE
    return pl.pallas_call(
        _edge_stats_body,
        grid=(steps,),
        in_specs=_edge_in_specs(),
        out_specs=pl.BlockSpec((2, 2 * _D), lambda i: (0, 0)),
        out_shape=jax.ShapeDtypeStruct((2, 2 * _D), jnp.float32),
    )(gh, efb, wcat, wef)


def _tc_edge_apply(gh, efb, wcat, wef, k, t):
    steps = _E // _BE
    vec = pl.BlockSpec((1, 2 * _D), lambda i: (0, 0))
    return pl.pallas_call(
        _edge_apply_body,
        grid=(steps,),
        in_specs=_edge_in_specs() + [vec, vec],
        out_specs=[pl.BlockSpec((_BE, 32), lambda i: (i, 0)),
                   pl.BlockSpec((_BE, 32), lambda i: (i, 0))],
        out_shape=[jax.ShapeDtypeStruct((_E, 32), jnp.float32),
                   jax.ShapeDtypeStruct((_E, 32), jnp.float32)],
    )(gh, efb, wcat, wef, k, t)


def _tc_agg_stats(alo, ahi):
    steps = _N // _BN
    return pl.pallas_call(
        _agg_stats_body,
        grid=(steps,),
        in_specs=[pl.BlockSpec((_BN, 32), lambda i: (i, 0)),
                  pl.BlockSpec((_BN, 32), lambda i: (i, 0))],
        out_specs=pl.BlockSpec((2, _D), lambda i: (0, 0)),
        out_shape=jax.ShapeDtypeStruct((2, _D), jnp.float32),
    )(alo, ahi)


def _tc_update(h, alo, ahi, k, t):
    steps = _N // _BN
    return pl.pallas_call(
        _update_body,
        grid=(steps,),
        in_specs=[pl.BlockSpec((_BN, _D), lambda i: (i, 0)),
                  pl.BlockSpec((_BN, 32), lambda i: (i, 0)),
                  pl.BlockSpec((_BN, 32), lambda i: (i, 0)),
                  pl.BlockSpec((1, _D), lambda i: (0, 0)),
                  pl.BlockSpec((1, _D), lambda i: (0, 0))],
        out_specs=[pl.BlockSpec((_BN, _D), lambda i: (i, 0)),
                   pl.BlockSpec((1, _D), lambda i: (0, 0))],
        out_shape=[jax.ShapeDtypeStruct((_N, _D), jnp.float32),
                   jax.ShapeDtypeStruct((1, _D), jnp.float32)],
    )(h, alo, ahi, k, t)


def _tc_fc(hsum, wfc, bfc, wout, bout):
    return pl.pallas_call(
        _fc_body,
        out_shape=jax.ShapeDtypeStruct((1, 1), jnp.float32),
    )(hsum, wfc, bfc[None, :], wout, bout[None, :])


def _affine(g, be, sums, n):
    s, ss = sums[0], sums[1]
    mu = s / n
    var = ss / n - mu * mu
    rstd = 1.0 / jnp.sqrt(var + _EPS)
    k = g * rstd
    t = be - g * mu * rstd
    return k[None, :], t[None, :]


def kernel(atom_features, bondlength, W_emb, b_emb, W_int, b_int, g_int,
           be_int, W_upd, b_upd, g_upd, be_upd, g_bn, be_bn, W_fc, b_fc,
           W_out, b_out, edge_index):
    ei = edge_index.astype(jnp.int32)
    src = ei[0]
    dst = ei[1]
    bond2d = bondlength.reshape(_E, 1)
    cen = jnp.asarray(_CENTERS)[None, :]

    h, hsum = _tc_embed(atom_features, W_emb, b_emb[None, :])
    efb = _tc_ef(bond2d, cen)

    bf = jnp.bfloat16
    L = W_int.shape[0]
    for l in range(L):
        wcat = jnp.concatenate(
            [jnp.concatenate([W_int[l, :_D], W_upd[l, :_D]], axis=1),
             jnp.concatenate([W_int[l, _D:2 * _D], W_upd[l, _D:2 * _D]], axis=1)],
            axis=0).astype(bf)
        wef = jnp.concatenate([W_int[l, 2 * _D:], W_upd[l, 2 * _D:]],
                              axis=1).astype(bf)

        hb = _tc_center(h, hsum * (1.0 / _N))
        gh = _sc_gather(hb, src, dst)
        stats = _tc_edge_stats(gh, efb, wcat, wef)
        k_cat, t_cat = _affine(jnp.concatenate([g_int[l], g_upd[l]]),
                               jnp.concatenate([be_int[l], be_upd[l]]),
                               stats, float(_E))
        mlo, mhi = _tc_edge_apply(gh, efb, wcat, wef, k_cat, t_cat)
        alo, ahi = _sc_scatter(mlo, mhi, dst)
        astats = _tc_agg_stats(alo, ahi)
        k_bn, t_bn = _affine(g_bn[l], be_bn[l], astats, float(_N))
        h, hsum = _tc_update(h, alo, ahi, k_bn, t_bn)

    out = _tc_fc(hsum, W_fc, b_fc, W_out, b_out)
    return jnp.squeeze(out)


# speedup vs baseline: 3.8241x; 1.9468x over previous
import functools

import numpy as np
import jax
import jax.numpy as jnp
from jax import lax
from jax.experimental import pallas as pl
from jax.experimental.pallas import tpu as pltpu
from jax.experimental.pallas import tpu_sc as plsc

_N = 50000
_E = 800000
_D = 64
_DE = 16
_EPS = 1e-5

_BE = 6400
_EP = 819200
_EFB = 8192
_BN = 2000

_NC = 2
_NS = 16
_NW = _NC * _NS
_GC = 1000
_MC = 400
_SN = _N // _NS
_ZR = 125

_CENTERS = np.linspace(0.0, 8.0, _DE, dtype=np.float32)
_GAMMA = float(1.0 / np.diff(_CENTERS).mean())

@functools.lru_cache(maxsize=None)
def _sc_mesh():
    return plsc.VectorSubcoreMesh(core_axis_name="c", subcore_axis_name="s",
                                  num_cores=_NC, num_subcores=_NS)


def _softplus(x):
    return jnp.maximum(x, 0.0) + jnp.log(1.0 + jnp.exp(-jnp.abs(x)))


def _sigmoid(x):
    return 1.0 / (1.0 + jnp.exp(-x))


def _dot(a, b):
    return jax.lax.dot_general(a, b, (((1,), (0,)), ((), ())),
                               preferred_element_type=jnp.float32)


def _embed_body(a_ref, w_ref, b_ref, o_ref):
    o_ref[...] = _dot(a_ref[...], w_ref[...]) + b_ref[...]


def _ef_body(bond_ref, cen_ref, ef_ref):
    b = bond_ref[...].reshape(1, _EFB)
    d = b - cen_ref[...]
    ef_ref[...] = jnp.exp(-_GAMMA * d * d)


def _dot_t(a, b):
    return jax.lax.dot_general(a, b, (((0,), (0,)), ((), ())),
                               preferred_element_type=jnp.float32)


def _edge_x(gh, eft, wcat, wef):
    return _dot(gh[...], wcat[...]) + _dot_t(eft[...], wef[...])


def _edge_stats_body(gh, eft, wcat, wef, out_ref):
    x = _edge_x(gh, eft, wcat, wef)
    s = jnp.sum(x, 0, keepdims=True)
    ss = jnp.sum(x * x, 0, keepdims=True)
    val = jnp.concatenate([s, ss], axis=0)

    @pl.when(pl.program_id(0) == 0)
    def _():
        out_ref[...] = jnp.zeros_like(out_ref)

    out_ref[...] += val


def _edge_apply_body(gh_a, gh_b, eft_a, eft_b, wcat, wef, k_ref, t_ref,
                     m2_ref):
    def half(gh, eft):
        x = _edge_x(gh, eft, wcat, wef)
        y = x * k_ref[...] + t_ref[...]
        gate = _sigmoid(y[:, :_D])
        upd = _softplus(y[:, _D:])
        return gate * upd

    m2_ref[...] = jnp.concatenate([half(gh_a, eft_a), half(gh_b, eft_b)],
                                  axis=1)


def _agg_stats_body(alo, ahi, out_ref):
    a = jnp.concatenate([alo[...], ahi[...]], axis=1)
    s = jnp.sum(a, 0, keepdims=True)
    ss = jnp.sum(a * a, 0, keepdims=True)
    val = jnp.concatenate([s, ss], axis=0)

    @pl.when(pl.program_id(0) == 0)
    def _():
        out_ref[...] = jnp.zeros_like(out_ref)

    out_ref[...] += val


def _update_body(h_ref, alo, ahi, k_ref, t_ref, hn_ref, hsum_ref):
    a = jnp.concatenate([alo[...], ahi[...]], axis=1)
    hn = _softplus(h_ref[...] + a * k_ref[...] + t_ref[...])
    hn_ref[...] = hn

    @pl.when(pl.program_id(0) == 0)
    def _():
        hsum_ref[...] = jnp.zeros_like(hsum_ref)

    hsum_ref[...] += jnp.sum(hn, 0, keepdims=True)


def _fc_body(hsum_ref, wfc_ref, bfc_ref, wout_ref, bout_ref, out_ref):
    f = _softplus(hsum_ref[...] * (1.0 / _N))
    f = _softplus(_dot(f, wfc_ref[...]) + bfc_ref[...])
    f = _softplus(f)
    out_ref[...] = _dot(f, wout_ref[...]) + bout_ref[...]


def _sc_gather_body(h_hbm, src_hbm, dst_hbm, gh_hbm,
                    idx_s, idx_d, buf_s, buf_d, sem_s, sem_d):
    cid = lax.axis_index("c")
    sid = lax.axis_index("s")
    wid = sid * _NC + cid
    per_w = _E // _NW
    base = wid * per_w

    def step(i, carry):
        off = base + i * _GC
        pltpu.sync_copy(src_hbm.at[pl.ds(off, _GC)], idx_s)
        pltpu.sync_copy(dst_hbm.at[pl.ds(off, _GC)], idx_d)
        ca = pltpu.async_copy(h_hbm.at[idx_s], buf_s, sem_s)
        cb = pltpu.async_copy(h_hbm.at[idx_d], buf_d, sem_d)
        ca.wait()
        cb.wait()
        pltpu.sync_copy(buf_s, gh_hbm.at[pl.ds(off, _GC), pl.ds(0, _D)])
        pltpu.sync_copy(buf_d, gh_hbm.at[pl.ds(off, _GC), pl.ds(_D, _D)])
        return carry

    lax.fori_loop(0, per_w // _GC, step, 0)


@functools.lru_cache(maxsize=None)
def _sc_gather_kernel():
    return pl.kernel(
        _sc_gather_body,
        out_type=[jax.ShapeDtypeStruct((_E, 2 * _D), jnp.float32)],
        mesh=_sc_mesh(),
        scratch_types=[pltpu.VMEM((_GC,), jnp.int32),
                       pltpu.VMEM((_GC,), jnp.int32),
                       pltpu.VMEM((_GC, _D), jnp.float32),
                       pltpu.VMEM((_GC, _D), jnp.float32),
                       pltpu.SemaphoreType.DMA,
                       pltpu.SemaphoreType.DMA],
        compiler_params=pltpu.CompilerParams(use_tc_tiling_on_sc=False),
    )


def _sc_gather(h, src, dst):
    return _sc_gather_kernel()(h, src, dst)[0]


def _sc_scatter_body(m2_hbm, dste_hbm, dsto_hbm, olo_hbm, ohi_hbm,
                     idx_e, idx_o, buf_e, buf_o, zbuf, agg_sh):
    cid = lax.axis_index("c")
    sid = lax.axis_index("s")

    zv = jnp.zeros((16,), jnp.float32)

    def zstep(r, carry):
        zbuf[r, pl.ds(0, 16)] = zv
        zbuf[r, pl.ds(16, 16)] = zv
        return carry

    lax.fori_loop(0, _ZR, zstep, 0)

    def zcopy(j, carry):
        pltpu.sync_copy(zbuf, agg_sh.at[pl.ds(sid * _SN + j * _ZR, _ZR), :])
        return carry

    lax.fori_loop(0, _SN // _ZR, zcopy, 0)
    plsc.subcore_barrier()

    per_t = _E // _NS // 2
    hc = _MC // 2
    col = cid * 32

    def step(i, carry):
        off = sid * per_t + i * hc
        pltpu.sync_copy(dste_hbm.at[pl.ds(off, hc)], idx_e)
        pltpu.sync_copy(dsto_hbm.at[pl.ds(off, hc)], idx_o)
        pltpu.sync_copy(m2_hbm.at[pl.ds(off, hc), pl.ds(col, 32)], buf_e)
        pltpu.sync_copy(m2_hbm.at[pl.ds(off, hc), pl.ds(_D + col, 32)], buf_o)
        pltpu.sync_copy(buf_e, agg_sh.at[idx_e], add=True)
        pltpu.sync_copy(buf_o, agg_sh.at[idx_o], add=True)
        return carry

    lax.fori_loop(0, per_t // hc, step, 0)

    plsc.subcore_barrier()

    @pl.when(cid == 0)
    def _():
        pltpu.sync_copy(agg_sh.at[pl.ds(sid * _SN, _SN), :],
                        olo_hbm.at[pl.ds(sid * _SN, _SN), :])

    @pl.when(cid == 1)
    def _():
        pltpu.sync_copy(agg_sh.at[pl.ds(sid * _SN, _SN), :],
                        ohi_hbm.at[pl.ds(sid * _SN, _SN), :])


@functools.lru_cache(maxsize=None)
def _sc_scatter_kernel():
    return pl.kernel(
        _sc_scatter_body,
        out_type=[jax.ShapeDtypeStruct((_N, 32), jnp.float32),
                  jax.ShapeDtypeStruct((_N, 32), jnp.float32)],
        mesh=_sc_mesh(),
        scratch_types=[pltpu.VMEM((_MC // 2,), jnp.int32),
                       pltpu.VMEM((_MC // 2,), jnp.int32),
                       pltpu.VMEM((_MC // 2, 32), jnp.float32),
                       pltpu.VMEM((_MC // 2, 32), jnp.float32),
                       pltpu.VMEM((_ZR, 32), jnp.float32),
                       pltpu.VMEM_SHARED((_N, 32), jnp.float32)],
        compiler_params=pltpu.CompilerParams(use_tc_tiling_on_sc=False),
    )


def _sc_scatter(m2, dst_e, dst_o):
    return _sc_scatter_kernel()(m2, dst_e, dst_o)


def _tc_embed(atom, w, b):
    steps = _N // _BN
    return pl.pallas_call(
        _embed_body,
        grid=(steps,),
        in_specs=[pl.BlockSpec((_BN, 92), lambda i: (i, 0)),
                  pl.BlockSpec((92, _D), lambda i: (0, 0)),
                  pl.BlockSpec((1, _D), lambda i: (0, 0))],
        out_specs=pl.BlockSpec((_BN, _D), lambda i: (i, 0)),
        out_shape=jax.ShapeDtypeStruct((_N, _D), jnp.float32),
    )(atom, w, b)


def _tc_ef(bond2, cen):
    steps = _EP // _EFB
    return pl.pallas_call(
        _ef_body,
        grid=(steps,),
        in_specs=[pl.BlockSpec((8, 1024), lambda i: (i, 0)),
                  pl.BlockSpec((_DE, 1), lambda i: (0, 0))],
        out_specs=pl.BlockSpec((_DE, _EFB), lambda i: (0, i)),
        out_shape=jax.ShapeDtypeStruct((_DE, _EP), jnp.float32),
    )(bond2, cen)


def _edge_in_specs():
    return [pl.BlockSpec((_BE, 2 * _D), lambda i: (i, 0)),
            pl.BlockSpec((_DE, _BE), lambda i: (0, i)),
            pl.BlockSpec((2 * _D, 2 * _D), lambda i: (0, 0)),
            pl.BlockSpec((_DE, 2 * _D), lambda i: (0, 0))]


def _tc_edge_stats(gh, eft, wcat, wef):
    steps = _E // _BE
    return pl.pallas_call(
        _edge_stats_body,
        grid=(steps,),
        in_specs=_edge_in_specs(),
        out_specs=pl.BlockSpec((2, 2 * _D), lambda i: (0, 0)),
        out_shape=jax.ShapeDtypeStruct((2, 2 * _D), jnp.float32),
    )(gh, eft, wcat, wef)


def _tc_edge_apply(gh, eft, wcat, wef, k, t):
    hb = _BE // 2
    steps = _E // _BE
    vec = pl.BlockSpec((1, 2 * _D), lambda i: (0, 0))
    return pl.pallas_call(
        _edge_apply_body,
        grid=(steps,),
        in_specs=[pl.BlockSpec((hb, 2 * _D), lambda i: (i, 0)),
                  pl.BlockSpec((hb, 2 * _D), lambda i: (i + _E // _BE, 0)),
                  pl.BlockSpec((_DE, hb), lambda i: (0, i)),
                  pl.BlockSpec((_DE, hb), lambda i: (0, i + _E // _BE)),
                  pl.BlockSpec((2 * _D, 2 * _D), lambda i: (0, 0)),
                  pl.BlockSpec((_DE, 2 * _D), lambda i: (0, 0)),
                  vec, vec],
        out_specs=pl.BlockSpec((hb, 2 * _D), lambda i: (i, 0)),
        out_shape=jax.ShapeDtypeStruct((_E // 2, 2 * _D), jnp.float32),
    )(gh, gh, eft, eft, wcat, wef, k, t)


def _tc_agg_stats(alo, ahi):
    steps = _N // _BN
    return pl.pallas_call(
        _agg_stats_body,
        grid=(steps,),
        in_specs=[pl.BlockSpec((_BN, 32), lambda i: (i, 0)),
                  pl.BlockSpec((_BN, 32), lambda i: (i, 0))],
        out_specs=pl.BlockSpec((2, _D), lambda i: (0, 0)),
        out_shape=jax.ShapeDtypeStruct((2, _D), jnp.float32),
    )(alo, ahi)


def _tc_update(h, alo, ahi, k, t):
    steps = _N // _BN
    return pl.pallas_call(
        _update_body,
        grid=(steps,),
        in_specs=[pl.BlockSpec((_BN, _D), lambda i: (i, 0)),
                  pl.BlockSpec((_BN, 32), lambda i: (i, 0)),
                  pl.BlockSpec((_BN, 32), lambda i: (i, 0)),
                  pl.BlockSpec((1, _D), lambda i: (0, 0)),
                  pl.BlockSpec((1, _D), lambda i: (0, 0))],
        out_specs=[pl.BlockSpec((_BN, _D), lambda i: (i, 0)),
                   pl.BlockSpec((1, _D), lambda i: (0, 0))],
        out_shape=[jax.ShapeDtypeStruct((_N, _D), jnp.float32),
                   jax.ShapeDtypeStruct((1, _D), jnp.float32)],
    )(h, alo, ahi, k, t)


def _tc_fc(hsum, wfc, bfc, wout, bout):
    return pl.pallas_call(
        _fc_body,
        out_shape=jax.ShapeDtypeStruct((1, 1), jnp.float32),
    )(hsum, wfc, bfc[None, :], wout, bout[None, :])


def _affine(g, be, sums, n):
    s, ss = sums[0], sums[1]
    mu = s / n
    var = ss / n - mu * mu
    rstd = 1.0 / jnp.sqrt(var + _EPS)
    k = g * rstd
    t = be - g * mu * rstd
    return k[None, :], t[None, :]


def kernel(atom_features, bondlength, W_emb, b_emb, W_int, b_int, g_int,
           be_int, W_upd, b_upd, g_upd, be_upd, g_bn, be_bn, W_fc, b_fc,
           W_out, b_out, edge_index):
    ei = edge_index.astype(jnp.int32)
    src = ei[0]
    dst = ei[1]
    dst_e = dst[:_E // 2]
    dst_o = dst[_E // 2:]
    bond2 = jnp.concatenate(
        [bondlength, jnp.zeros((_EP - _E,), jnp.float32)]).reshape(
            _EP // 1024, 1024)
    cen = jnp.asarray(_CENTERS)[:, None]

    h = _tc_embed(atom_features, W_emb, b_emb[None, :])
    eft = _tc_ef(bond2, cen)

    L = W_int.shape[0]
    hsum = None
    for l in range(L):
        wcat = jnp.concatenate(
            [jnp.concatenate([W_int[l, :_D], W_upd[l, :_D]], axis=1),
             jnp.concatenate([W_int[l, _D:2 * _D], W_upd[l, _D:2 * _D]], axis=1)],
            axis=0)
        wef = jnp.concatenate([W_int[l, 2 * _D:], W_upd[l, 2 * _D:]], axis=1)

        gh = _sc_gather(h, src, dst)
        stats = _tc_edge_stats(gh, eft, wcat, wef)
        k_cat, t_cat = _affine(jnp.concatenate([g_int[l], g_upd[l]]),
                               jnp.concatenate([be_int[l], be_upd[l]]),
                               stats, float(_E))
        m2 = _tc_edge_apply(gh, eft, wcat, wef, k_cat, t_cat)
        alo, ahi = _sc_scatter(m2, dst_e, dst_o)
        astats = _tc_agg_stats(alo, ahi)
        k_bn, t_bn = _affine(g_bn[l], be_bn[l], astats, float(_N))
        h, hsum = _tc_update(h, alo, ahi, k_bn, t_bn)

    out = _tc_fc(hsum, W_fc, b_fc, W_out, b_out)
    return jnp.squeeze(out)
